# baseline (speedup 1.0000x reference)
import functools

import jax
import jax.numpy as jnp
from jax import lax
from jax.experimental import pallas as pl
from jax.experimental.pallas import tpu as pltpu
from jax.experimental.pallas import tpu_sc as plsc

_B, _S = 4096, 26
_F = 32
_TOTAL = _B * _S
_NC, _NS = 2, 16
_NW = _NC * _NS
_PER_W = _TOTAL // _NW
_CHUNK = 128
_NCHUNK = _PER_W // _CHUNK

_mesh = plsc.VectorSubcoreMesh(core_axis_name="c", subcore_axis_name="s")


@functools.partial(
    pl.kernel,
    out_type=jax.ShapeDtypeStruct((_S, _F, _B), jnp.float32),
    mesh=_mesh,
    scratch_types=[
        pltpu.VMEM((_NCHUNK, _CHUNK), jnp.int32),
        pltpu.VMEM((_PER_W, _F), jnp.float32),
        pltpu.VMEM((2, _F, _CHUNK), jnp.float32),
        pltpu.SemaphoreType.DMA,
        pltpu.SemaphoreType.DMA,
    ],
    compiler_params=pltpu.CompilerParams(
        use_tc_tiling_on_sc=False, needs_layout_passes=False
    ),
)
def _gather_kernel(idx_hbm, table_hbm, out_hbm, idx_v, rows_v, rowsT, gsem, osem):
    wid = lax.axis_index("s") * _NC + lax.axis_index("c")
    pltpu.sync_copy(idx_hbm.at[pl.ds(wid * _NCHUNK, _NCHUNK)], idx_v)
    for j in range(_NCHUNK):
        pltpu.async_copy(
            table_hbm.at[idx_v.at[j]],
            rows_v.at[pl.ds(j * _CHUNK, _CHUNK)],
            gsem,
        )
    lanes = lax.iota(jnp.int32, 16)

    def per_chunk(j, carry):
        pltpu.make_async_copy(
            table_hbm.at[idx_v.at[0]],
            rows_v.at[pl.ds(j * _CHUNK, _CHUNK)],
            gsem,
        ).wait()

        buf = rowsT.at[j & 1]

        @pl.when(j >= 2)
        def _():
            pltpu.make_async_copy(buf, out_hbm.at[0, :, pl.ds(0, _CHUNK)], osem).wait()

        for bg in range(_CHUNK // 16):
            row_ids = lanes + (j * _CHUNK + bg * 16)
            for f in range(_F):
                v = plsc.load_gather(
                    rows_v, [row_ids, jnp.full((16,), f, jnp.int32)]
                )
                buf[f, pl.ds(bg * 16, 16)] = v

        u = wid * _NCHUNK + j
        s = u >> 5
        bblk = u & 31
        pltpu.async_copy(
            buf, out_hbm.at[s, :, pl.ds(bblk * _CHUNK, _CHUNK)], osem
        )
        return carry

    lax.fori_loop(0, _NCHUNK, per_chunk, 0)
    for _ in range(2):
        pltpu.make_async_copy(
            rowsT.at[0], out_hbm.at[0, :, pl.ds(0, _CHUNK)], osem
        ).wait()


_TBLK = 4096
_TGRID = -(-1000000 // _TBLK)


def _transpose_body(x_ref, o_ref):
    o_ref[...] = x_ref[...].T


_tc_transpose = pl.pallas_call(
    _transpose_body,
    out_shape=jax.ShapeDtypeStruct((1000000, _F), jnp.float32),
    grid=(_TGRID,),
    in_specs=[pl.BlockSpec((_F, _TBLK), lambda j: (0, j))],
    out_specs=pl.BlockSpec((_TBLK, _F), lambda j: (j, 0)),
    compiler_params=pltpu.CompilerParams(
        dimension_semantics=("arbitrary",),
    ),
)


def kernel(inputs, embedding):
    idx = inputs.T.reshape(_NW * _NCHUNK, _CHUNK)
    table_rm = _tc_transpose(embedding.T)
    out = _gather_kernel(idx, table_rm)
    return out.transpose(2, 0, 1)


# speedup vs baseline: 1.2738x; 1.2738x over previous
import functools

import jax
import jax.numpy as jnp
from jax import lax
from jax.experimental import pallas as pl
from jax.experimental.pallas import tpu as pltpu
from jax.experimental.pallas import tpu_sc as plsc

_B, _S = 4096, 26
_F = 32
_TOTAL = _B * _S
_NC, _NS = 2, 16
_NW = _NC * _NS
_PER_W = _TOTAL // _NW
_CHUNK = 128
_NCHUNK = _PER_W // _CHUNK
_QROWS = 250000

_mesh = plsc.VectorSubcoreMesh(core_axis_name="c", subcore_axis_name="s")


@functools.partial(
    pl.kernel,
    out_type=jax.ShapeDtypeStruct((_S, _F, _B), jnp.float32),
    mesh=_mesh,
    scratch_types=[
        pltpu.VMEM((_NCHUNK, _CHUNK), jnp.int32),
        pltpu.VMEM((_NCHUNK, _CHUNK), jnp.int32),
        pltpu.VMEM((_NCHUNK, _CHUNK), jnp.int32),
        pltpu.VMEM((2, _CHUNK, _CHUNK), jnp.float32),
        pltpu.VMEM((2, _F, _CHUNK), jnp.float32),
        pltpu.SemaphoreType.DMA,
        pltpu.SemaphoreType.DMA,
    ],
    compiler_params=pltpu.CompilerParams(needs_layout_passes=False),
)
def _gather_kernel(
    idx_hbm, table_hbm, out_hbm, idx_v, g_v, q_v, quads, rowsT, gsem, osem
):
    wid = lax.axis_index("s") * _NC + lax.axis_index("c")
    pltpu.sync_copy(idx_hbm.at[wid], idx_v)
    for j in range(_NCHUNK):
        for bg in range(_CHUNK // 16):
            v = idx_v[j, pl.ds(bg * 16, 16)]
            g_v[j, pl.ds(bg * 16, 16)] = v >> 2
            q_v[j, pl.ds(bg * 16, 16)] = (v & 3) << 5
    for j in range(2):
        pltpu.async_copy(table_hbm.at[g_v.at[j]], quads.at[j & 1], gsem)

    lanes = lax.iota(jnp.int32, 16)

    def per_chunk(j, carry):
        pltpu.make_async_copy(
            table_hbm.at[g_v.at[0]], quads.at[0], gsem
        ).wait()
        qbuf = quads.at[j & 1]
        tbuf = rowsT.at[j & 1]

        @pl.when(j >= 2)
        def _():
            pltpu.make_async_copy(
                tbuf, out_hbm.at[0, :, pl.ds(0, _CHUNK)], osem
            ).wait()

        for bg in range(_CHUNK // 16):
            b_ids = lanes + bg * 16
            qb = q_v[j, pl.ds(bg * 16, 16)]
            for f in range(_F):
                v = plsc.load_gather(qbuf, [b_ids, qb + f])
                tbuf[f, pl.ds(bg * 16, 16)] = v

        @pl.when(j + 2 < _NCHUNK)
        def _():
            pltpu.async_copy(table_hbm.at[g_v.at[j + 2]], qbuf, gsem)

        u = wid * _NCHUNK + j
        s = u >> 5
        bblk = u & 31
        pltpu.async_copy(
            tbuf, out_hbm.at[s, :, pl.ds(bblk * _CHUNK, _CHUNK)], osem
        )
        return carry

    lax.fori_loop(0, _NCHUNK, per_chunk, 0)
    for _ in range(2):
        pltpu.make_async_copy(
            rowsT.at[0], out_hbm.at[0, :, pl.ds(0, _CHUNK)], osem
        ).wait()


def kernel(inputs, embedding):
    idx = inputs.T.reshape(_NW, _NCHUNK, _CHUNK)
    table_q = embedding.reshape(_QROWS, _CHUNK)
    out = _gather_kernel(idx, table_q)
    return out.transpose(2, 0, 1)
